# baseline (speedup 1.0000x reference)
import functools

import jax
import jax.numpy as jnp
from jax import lax
from jax.experimental import pallas as pl
from jax.experimental.pallas import tpu as pltpu
from jax.experimental.pallas import tpu_sc as plsc

BATCH = 16384
EMBED_DIM = 32
NUM_WORKERS = 32
B_PER_W = BATCH // NUM_WORKERS
CHUNK = 128
NCHUNK = B_PER_W // CHUNK

MLP_BLK = 2048


def _gather_body(uid_hbm, iid_hbm, uemb_hbm, iemb_hbm, u_out, v_out,
                 uidx_v, iidx_v, urows_v, irows_v, sem):
    wid = lax.axis_index("s") * 2 + lax.axis_index("c")
    base = wid * B_PER_W
    copies = []
    for j in range(NCHUNK):
        off = base + j * CHUNK
        pltpu.sync_copy(uid_hbm.at[pl.ds(off, CHUNK)], uidx_v.at[j])
        pltpu.sync_copy(iid_hbm.at[pl.ds(off, CHUNK)], iidx_v.at[j])
    for j in range(NCHUNK):
        copies.append(pltpu.async_copy(uemb_hbm.at[uidx_v.at[j]], urows_v.at[j], sem))
        copies.append(pltpu.async_copy(iemb_hbm.at[iidx_v.at[j]], irows_v.at[j], sem))
    for c in copies:
        c.wait()
    for j in range(NCHUNK):
        off = base + j * CHUNK
        pltpu.sync_copy(urows_v.at[j], u_out.at[pl.ds(off, CHUNK)])
        pltpu.sync_copy(irows_v.at[j], v_out.at[pl.ds(off, CHUNK)])


@functools.cache
def _gather_call():
    return functools.partial(
        pl.kernel,
        out_type=(
            jax.ShapeDtypeStruct((BATCH, EMBED_DIM), jnp.float32),
            jax.ShapeDtypeStruct((BATCH, EMBED_DIM), jnp.float32),
        ),
        mesh=plsc.VectorSubcoreMesh(core_axis_name="c", subcore_axis_name="s"),
        scratch_types=[
            pltpu.VMEM((NCHUNK, CHUNK), jnp.int32),
            pltpu.VMEM((NCHUNK, CHUNK), jnp.int32),
            pltpu.VMEM((NCHUNK, CHUNK, EMBED_DIM), jnp.float32),
            pltpu.VMEM((NCHUNK, CHUNK, EMBED_DIM), jnp.float32),
            pltpu.SemaphoreType.DMA,
        ],
        compiler_params=pltpu.CompilerParams(use_tc_tiling_on_sc=False),
    )(_gather_body)


def _mlp_body(u_ref, v_ref, w1u_ref, w1v_ref, b1_ref, w2_ref, b2_ref,
              w3_ref, b3_ref, w4t_ref, b4_ref, out_ref):
    x = jnp.dot(u_ref[...], w1u_ref[...], preferred_element_type=jnp.float32)
    x = x + jnp.dot(v_ref[...], w1v_ref[...], preferred_element_type=jnp.float32)
    h = jnp.maximum(x + b1_ref[...], 0.0)
    h = jnp.maximum(
        jnp.dot(h, w2_ref[...], preferred_element_type=jnp.float32) + b2_ref[...], 0.0)
    h = jnp.maximum(
        jnp.dot(h, w3_ref[...], preferred_element_type=jnp.float32) + b3_ref[...], 0.0)
    out_ref[...] = jnp.sum(h * w4t_ref[...], axis=1) + b4_ref[0, 0]


def _full(shape):
    return pl.BlockSpec(shape, lambda i: tuple(0 for _ in shape))


_mlp_call = pl.pallas_call(
    _mlp_body,
    grid=(BATCH // MLP_BLK,),
    in_specs=[
        pl.BlockSpec((MLP_BLK, EMBED_DIM), lambda i: (i, 0)),
        pl.BlockSpec((MLP_BLK, EMBED_DIM), lambda i: (i, 0)),
        _full((EMBED_DIM, 128)),
        _full((EMBED_DIM, 128)),
        _full((1, 128)),
        _full((128, 64)),
        _full((1, 64)),
        _full((64, 32)),
        _full((1, 32)),
        _full((1, 32)),
        _full((1, 1)),
    ],
    out_specs=pl.BlockSpec((MLP_BLK,), lambda i: (i,)),
    out_shape=jax.ShapeDtypeStruct((BATCH,), jnp.float32),
)


def kernel(user_ids, item_ids, user_emb, item_emb, W1, b1, W2, b2, W3, b3, W4, b4):
    u, v = _gather_call()(user_ids.astype(jnp.int32), item_ids.astype(jnp.int32),
                          user_emb, item_emb)
    return _mlp_call(
        u, v,
        W1[:EMBED_DIM], W1[EMBED_DIM:], b1.reshape(1, 128),
        W2, b2.reshape(1, 64),
        W3, b3.reshape(1, 32),
        W4.reshape(1, 32), b4.reshape(1, 1),
    )


# speedup vs baseline: 1.5592x; 1.5592x over previous
import functools

import jax
import jax.numpy as jnp
from jax import lax
from jax.experimental import pallas as pl
from jax.experimental.pallas import tpu as pltpu
from jax.experimental.pallas import tpu_sc as plsc

BATCH = 16384
EMBED_DIM = 32
NUM_WORKERS = 32
B_PER_W = BATCH // NUM_WORKERS
UNROLL = 16

MLP_BLK = 2048


def _fire_row_dmas(table_hbm, idx_v, rows_v, sem):
    def fire(c, carry):
        vec = idx_v[pl.ds(c * UNROLL, UNROLL)]
        for k in range(UNROLL):
            r = c * UNROLL + k
            pltpu.async_copy(table_hbm.at[pl.ds(vec[k], 1)],
                             rows_v.at[pl.ds(r, 1)], sem)
        return carry
    lax.fori_loop(0, B_PER_W // UNROLL, fire, 0, unroll=False)


def _gather_body(uid_hbm, iid_hbm, uemb_hbm, iemb_hbm, u_out, v_out,
                 uidx_v, iidx_v, rows_v, sem):
    wid = lax.axis_index("s") * 2 + lax.axis_index("c")
    base = wid * B_PER_W
    pltpu.sync_copy(uid_hbm.at[pl.ds(base, B_PER_W)], uidx_v)
    pltpu.sync_copy(iid_hbm.at[pl.ds(base, B_PER_W)], iidx_v)
    for table, idx_v, out in ((uemb_hbm, uidx_v, u_out), (iemb_hbm, iidx_v, v_out)):
        _fire_row_dmas(table, idx_v, rows_v, sem)
        pltpu.make_async_copy(table.at[pl.ds(0, B_PER_W)], rows_v, sem).wait()
        pltpu.sync_copy(rows_v, out.at[pl.ds(base, B_PER_W)])


@functools.cache
def _gather_call():
    return functools.partial(
        pl.kernel,
        out_type=(
            jax.ShapeDtypeStruct((BATCH, EMBED_DIM), jnp.float32),
            jax.ShapeDtypeStruct((BATCH, EMBED_DIM), jnp.float32),
        ),
        mesh=plsc.VectorSubcoreMesh(core_axis_name="c", subcore_axis_name="s"),
        scratch_types=[
            pltpu.VMEM((B_PER_W,), jnp.int32),
            pltpu.VMEM((B_PER_W,), jnp.int32),
            pltpu.VMEM((B_PER_W, EMBED_DIM), jnp.float32),
            pltpu.SemaphoreType.DMA,
        ],
    )(_gather_body)


def _mlp_body(u_ref, v_ref, w1u_ref, w1v_ref, b1_ref, w2_ref, b2_ref,
              w3_ref, b3_ref, w4t_ref, b4_ref, out_ref):
    x = jnp.dot(u_ref[...], w1u_ref[...], preferred_element_type=jnp.float32)
    x = x + jnp.dot(v_ref[...], w1v_ref[...], preferred_element_type=jnp.float32)
    h = jnp.maximum(x + b1_ref[...], 0.0)
    h = jnp.maximum(
        jnp.dot(h, w2_ref[...], preferred_element_type=jnp.float32) + b2_ref[...], 0.0)
    h = jnp.maximum(
        jnp.dot(h, w3_ref[...], preferred_element_type=jnp.float32) + b3_ref[...], 0.0)
    out_ref[...] = jnp.sum(h * w4t_ref[...], axis=1) + b4_ref[0, 0]


def _full(shape):
    return pl.BlockSpec(shape, lambda i: tuple(0 for _ in shape))


_mlp_call = pl.pallas_call(
    _mlp_body,
    grid=(BATCH // MLP_BLK,),
    in_specs=[
        pl.BlockSpec((MLP_BLK, EMBED_DIM), lambda i: (i, 0)),
        pl.BlockSpec((MLP_BLK, EMBED_DIM), lambda i: (i, 0)),
        _full((EMBED_DIM, 128)),
        _full((EMBED_DIM, 128)),
        _full((1, 128)),
        _full((128, 64)),
        _full((1, 64)),
        _full((64, 32)),
        _full((1, 32)),
        _full((1, 32)),
        _full((1, 1)),
    ],
    out_specs=pl.BlockSpec((MLP_BLK,), lambda i: (i,)),
    out_shape=jax.ShapeDtypeStruct((BATCH,), jnp.float32),
)


def kernel(user_ids, item_ids, user_emb, item_emb, W1, b1, W2, b2, W3, b3, W4, b4):
    u, v = _gather_call()(user_ids.astype(jnp.int32), item_ids.astype(jnp.int32),
                          user_emb, item_emb)
    return _mlp_call(
        u, v,
        W1[:EMBED_DIM], W1[EMBED_DIM:], b1.reshape(1, 128),
        W2, b2.reshape(1, 64),
        W3, b3.reshape(1, 32),
        W4.reshape(1, 32), b4.reshape(1, 1),
    )


# speedup vs baseline: 1.5617x; 1.0016x over previous
import functools

import jax
import jax.numpy as jnp
from jax import lax
from jax.experimental import pallas as pl
from jax.experimental.pallas import tpu as pltpu
from jax.experimental.pallas import tpu_sc as plsc

BATCH = 16384
EMBED_DIM = 32
NUM_WORKERS = 32
B_PER_W = BATCH // NUM_WORKERS
UNROLL = 16

MLP_BLK = 2048


def _fire_row_dmas(table_hbm, idx_v, rows_v, sem):
    def fire(c, carry):
        vec = idx_v[pl.ds(c * UNROLL, UNROLL)]
        for k in range(UNROLL):
            r = c * UNROLL + k
            pltpu.async_copy(table_hbm.at[pl.ds(vec[k], 1)],
                             rows_v.at[pl.ds(r, 1)], sem)
        return carry
    lax.fori_loop(0, B_PER_W // UNROLL, fire, 0, unroll=False)


def _gather_body(uid_hbm, iid_hbm, uemb_hbm, iemb_hbm, u_out, v_out,
                 uidx_v, iidx_v, rows_v, sem):
    wid = lax.axis_index("s") * 2 + lax.axis_index("c")
    base = wid * B_PER_W
    pltpu.sync_copy(uid_hbm.at[pl.ds(base, B_PER_W)], uidx_v)
    pltpu.sync_copy(iid_hbm.at[pl.ds(base, B_PER_W)], iidx_v)
    for table, idx_v, out in ((uemb_hbm, uidx_v, u_out), (iemb_hbm, iidx_v, v_out)):
        _fire_row_dmas(table, idx_v, rows_v, sem)
        pltpu.make_async_copy(table.at[pl.ds(0, B_PER_W)], rows_v, sem).wait()
        pltpu.sync_copy(rows_v, out.at[pl.ds(base, B_PER_W)])


@functools.cache
def _gather_call():
    return functools.partial(
        pl.kernel,
        out_type=(
            jax.ShapeDtypeStruct((BATCH, EMBED_DIM), jnp.float32),
            jax.ShapeDtypeStruct((BATCH, EMBED_DIM), jnp.float32),
        ),
        mesh=plsc.VectorSubcoreMesh(core_axis_name="c", subcore_axis_name="s"),
        scratch_types=[
            pltpu.VMEM((B_PER_W,), jnp.int32),
            pltpu.VMEM((B_PER_W,), jnp.int32),
            pltpu.VMEM((B_PER_W, EMBED_DIM), jnp.float32),
            pltpu.SemaphoreType.DMA,
        ],
    )(_gather_body)


def _mlp_body(u_ref, v_ref, w1_ref, b1_ref, w2_ref, b2_ref,
              w3_ref, b3_ref, w4_ref, b4_ref, out_ref):
    w1 = w1_ref[...]
    x = jnp.dot(u_ref[...], w1[:EMBED_DIM], preferred_element_type=jnp.float32)
    x = x + jnp.dot(v_ref[...], w1[EMBED_DIM:], preferred_element_type=jnp.float32)
    h = jnp.maximum(x + b1_ref[...].reshape(1, 128), 0.0)
    h = jnp.maximum(
        jnp.dot(h, w2_ref[...], preferred_element_type=jnp.float32)
        + b2_ref[...].reshape(1, 64), 0.0)
    h = jnp.maximum(
        jnp.dot(h, w3_ref[...], preferred_element_type=jnp.float32)
        + b3_ref[...].reshape(1, 32), 0.0)
    w4t = w4_ref[...].reshape(1, 32)
    out_ref[...] = jnp.sum(h * w4t, axis=1) + b4_ref[...]


def _full(shape):
    return pl.BlockSpec(shape, lambda i: tuple(0 for _ in shape))


_mlp_call = pl.pallas_call(
    _mlp_body,
    grid=(BATCH // MLP_BLK,),
    in_specs=[
        pl.BlockSpec((MLP_BLK, EMBED_DIM), lambda i: (i, 0)),
        pl.BlockSpec((MLP_BLK, EMBED_DIM), lambda i: (i, 0)),
        _full((2 * EMBED_DIM, 128)),
        _full((128,)),
        _full((128, 64)),
        _full((64,)),
        _full((64, 32)),
        _full((32,)),
        _full((32, 1)),
        _full((1,)),
    ],
    out_specs=pl.BlockSpec((MLP_BLK,), lambda i: (i,)),
    out_shape=jax.ShapeDtypeStruct((BATCH,), jnp.float32),
)


def kernel(user_ids, item_ids, user_emb, item_emb, W1, b1, W2, b2, W3, b3, W4, b4):
    u, v = _gather_call()(user_ids.astype(jnp.int32), item_ids.astype(jnp.int32),
                          user_emb, item_emb)
    return _mlp_call(u, v, W1, b1, W2, b2, W3, b3, W4, b4)
